# baseline (speedup 1.0000x reference)
import functools

import jax
import jax.numpy as jnp
from jax import lax
from jax.experimental import pallas as pl
from jax.experimental.pallas import tpu as pltpu
from jax.experimental.pallas import tpu_sc as plsc

NC = 2
NS = 16
NW = NC * NS
L = 16

D = 64
TCB = 8192
CHUNK = 64


@functools.lru_cache(maxsize=None)
def _make_tc_pack(n):
    grid = (n + TCB - 1) // TCB

    @functools.partial(
        pl.pallas_call,
        grid=(grid,),
        in_specs=[
            pl.BlockSpec((D, TCB), lambda i: (0, i)),
            pl.BlockSpec((D, TCB), lambda i: (0, i)),
        ],
        out_specs=pl.BlockSpec((TCB, 2 * D), lambda i: (i, 0)),
        out_shape=jax.ShapeDtypeStruct((grid * TCB, 2 * D), jnp.float32),
        compiler_params=pltpu.CompilerParams(
            dimension_semantics=("parallel",)),
    )
    def pack(a_ref, b_ref, o_ref):
        o_ref[:, 0:D] = a_ref[...].T
        o_ref[:, D:2 * D] = b_ref[...].T

    return pack


@functools.lru_cache(maxsize=None)
def _make_sc_scorer(total):
    assert total % (NW * 2 * CHUNK) == 0
    per_w = total // NW
    n_chunks = per_w // CHUNK
    mesh = plsc.VectorSubcoreMesh(core_axis_name="c", subcore_axis_name="s")

    rowbuf = pltpu.VMEM((CHUNK, 2 * D), jnp.float32)
    idxbuf = pltpu.VMEM((per_w,), jnp.int32)

    @functools.partial(
        pl.kernel,
        mesh=mesh,
        out_type=jax.ShapeDtypeStruct((total,), jnp.float32),
        compiler_params=pltpu.CompilerParams(needs_layout_passes=False),
        scratch_types=[
            idxbuf, idxbuf, idxbuf,
            [rowbuf] * 3,
            [rowbuf] * 3,
            pltpu.VMEM((per_w,), jnp.float32),
            pltpu.SemaphoreType.DMA,
            pltpu.SemaphoreType.DMA,
        ],
    )
    def scorer(h_hbm, t_hbm, r_hbm, ent_hbm, rel_hbm,
               out_hbm, hv, tv, rv, bufs0, bufs1, sv, sem0, sem1):
        wid = lax.axis_index("s") * NC + lax.axis_index("c")
        base = wid * per_w
        lanes = lax.iota(jnp.int32, L)
        bufsets = (bufs0, bufs1)
        sems = (sem0, sem1)

        pltpu.sync_copy(h_hbm.at[pl.ds(base, per_w)], hv)
        pltpu.sync_copy(t_hbm.at[pl.ds(base, per_w)], tv)
        pltpu.sync_copy(r_hbm.at[pl.ds(base, per_w)], rv)

        def issue(c, which):
            off = pl.ds(c * CHUNK, CHUNK)
            bufs, sem = bufsets[which], sems[which]
            return [
                pltpu.async_copy(ent_hbm.at[hv.at[off]], bufs[0], sem),
                pltpu.async_copy(ent_hbm.at[tv.at[off]], bufs[1], sem),
                pltpu.async_copy(rel_hbm.at[rv.at[off]], bufs[2], sem),
            ]

        def compute(c, which):
            hb, tb, rb = bufsets[which]

            def group_body(g, carry):
                i0 = g * L
                rows = lanes + i0
                acc = jnp.zeros((L,), jnp.float32)
                for d in range(D):
                    dv = jnp.full((L,), d, jnp.int32)
                    dv2 = jnp.full((L,), D + d, jnp.int32)
                    hhd = plsc.load_gather(hb, [rows, dv])
                    htd = plsc.load_gather(hb, [rows, dv2])
                    thd = plsc.load_gather(tb, [rows, dv])
                    ttd = plsc.load_gather(tb, [rows, dv2])
                    rhd = plsc.load_gather(rb, [rows, dv])
                    rtd = plsc.load_gather(rb, [rows, dv2])
                    acc = acc + (hhd * rhd * ttd + thd * rtd * htd)
                sv[pl.ds(c * CHUNK + i0, L)] = 0.5 * acc
                return carry

            lax.fori_loop(0, CHUNK // L, group_body, 0)

        def wait_chunk(c, which):
            off = pl.ds(c * CHUNK, CHUNK)
            bufs, sem = bufsets[which], sems[which]
            pltpu.make_async_copy(ent_hbm.at[hv.at[off]], bufs[0], sem).wait()
            pltpu.make_async_copy(ent_hbm.at[tv.at[off]], bufs[1], sem).wait()
            pltpu.make_async_copy(rel_hbm.at[rv.at[off]], bufs[2], sem).wait()

        issue(0, 0)

        def pair(k, carry):
            c0 = 2 * k
            issue(c0 + 1, 1)
            wait_chunk(c0, 0)
            compute(c0, 0)
            nxt = jnp.minimum(c0 + 2, n_chunks - 2)
            issue(nxt, 0)
            wait_chunk(c0 + 1, 1)
            compute(c0 + 1, 1)
            return carry

        lax.fori_loop(0, n_chunks // 2, pair, 0)
        wait_chunk(n_chunks - 2, 0)
        pltpu.sync_copy(sv, out_hbm.at[pl.ds(base, per_w)])

    return scorer


def kernel(pos_h, pos_r, pos_t, neg_h, neg_r, neg_t,
           entity_head, entity_tail, relation_head, relation_tail):
    b = pos_h.shape[0]
    h = jnp.concatenate([pos_h, neg_h])
    t = jnp.concatenate([pos_t, neg_t])
    r = jnp.concatenate([pos_r, neg_r])
    epack = _make_tc_pack(entity_head.shape[0])
    rpack = _make_tc_pack(relation_head.shape[0])
    ent = epack(entity_head.T, entity_tail.T)
    rel = rpack(relation_head.T, relation_tail.T)
    scorer = _make_sc_scorer(2 * b)
    out = scorer(h, t, r, ent, rel)
    return out[:b], out[b:]


# speedup vs baseline: 1.0361x; 1.0361x over previous
import functools

import jax
import jax.numpy as jnp
from jax import lax
from jax.experimental import pallas as pl
from jax.experimental.pallas import tpu as pltpu
from jax.experimental.pallas import tpu_sc as plsc

NC = 2
NS = 16
NW = NC * NS
L = 16

D = 64
TCB = 16384
CHUNK = 64


@functools.lru_cache(maxsize=None)
def _make_tc_pack(n):
    grid = (n + TCB - 1) // TCB

    @functools.partial(
        pl.pallas_call,
        grid=(grid,),
        in_specs=[
            pl.BlockSpec((D, TCB), lambda i: (0, i)),
            pl.BlockSpec((D, TCB), lambda i: (0, i)),
        ],
        out_specs=pl.BlockSpec((TCB, 2 * D), lambda i: (i, 0)),
        out_shape=jax.ShapeDtypeStruct((grid * TCB, 2 * D), jnp.float32),
        compiler_params=pltpu.CompilerParams(
            dimension_semantics=("parallel",)),
    )
    def pack(a_ref, b_ref, o_ref):
        o_ref[:, 0:D] = a_ref[...].T
        o_ref[:, D:2 * D] = b_ref[...].T

    return pack


@functools.lru_cache(maxsize=None)
def _make_sc_scorer(total):
    assert total % (NW * 2 * CHUNK) == 0
    per_w = total // NW
    n_chunks = per_w // CHUNK
    mesh = plsc.VectorSubcoreMesh(core_axis_name="c", subcore_axis_name="s")

    rowbuf = pltpu.VMEM((CHUNK, 2 * D), jnp.float32)
    idxbuf = pltpu.VMEM((per_w,), jnp.int32)

    @functools.partial(
        pl.kernel,
        mesh=mesh,
        out_type=jax.ShapeDtypeStruct((total,), jnp.float32),
        compiler_params=pltpu.CompilerParams(needs_layout_passes=False),
        scratch_types=[
            idxbuf, idxbuf, idxbuf,
            [rowbuf] * 3,
            [rowbuf] * 3,
            pltpu.VMEM((per_w,), jnp.float32),
            pltpu.SemaphoreType.DMA,
            pltpu.SemaphoreType.DMA,
        ],
    )
    def scorer(h_hbm, t_hbm, r_hbm, ent_hbm, rel_hbm,
               out_hbm, hv, tv, rv, bufs0, bufs1, sv, sem0, sem1):
        wid = lax.axis_index("s") * NC + lax.axis_index("c")
        base = wid * per_w
        lanes = lax.iota(jnp.int32, L)
        bufsets = (bufs0, bufs1)
        sems = (sem0, sem1)

        pltpu.sync_copy(h_hbm.at[pl.ds(base, per_w)], hv)
        pltpu.sync_copy(t_hbm.at[pl.ds(base, per_w)], tv)
        pltpu.sync_copy(r_hbm.at[pl.ds(base, per_w)], rv)

        def issue(c, which):
            off = pl.ds(c * CHUNK, CHUNK)
            bufs, sem = bufsets[which], sems[which]
            return [
                pltpu.async_copy(ent_hbm.at[hv.at[off]], bufs[0], sem),
                pltpu.async_copy(ent_hbm.at[tv.at[off]], bufs[1], sem),
                pltpu.async_copy(rel_hbm.at[rv.at[off]], bufs[2], sem),
            ]

        def compute(c, which):
            hb, tb, rb = bufsets[which]

            def group_body(g, carry):
                i0 = g * L
                rows = lanes + i0
                acc = jnp.zeros((L,), jnp.float32)
                for d in range(D):
                    dv = jnp.full((L,), d, jnp.int32)
                    dv2 = jnp.full((L,), D + d, jnp.int32)
                    hhd = plsc.load_gather(hb, [rows, dv])
                    htd = plsc.load_gather(hb, [rows, dv2])
                    thd = plsc.load_gather(tb, [rows, dv])
                    ttd = plsc.load_gather(tb, [rows, dv2])
                    rhd = plsc.load_gather(rb, [rows, dv])
                    rtd = plsc.load_gather(rb, [rows, dv2])
                    acc = acc + (hhd * rhd * ttd + thd * rtd * htd)
                sv[pl.ds(c * CHUNK + i0, L)] = 0.5 * acc
                return carry

            lax.fori_loop(0, CHUNK // L, group_body, 0)

        def wait_chunk(c, which):
            off = pl.ds(c * CHUNK, CHUNK)
            bufs, sem = bufsets[which], sems[which]
            pltpu.make_async_copy(ent_hbm.at[hv.at[off]], bufs[0], sem).wait()
            pltpu.make_async_copy(ent_hbm.at[tv.at[off]], bufs[1], sem).wait()
            pltpu.make_async_copy(rel_hbm.at[rv.at[off]], bufs[2], sem).wait()

        issue(0, 0)

        def pair(k, carry):
            c0 = 2 * k
            issue(c0 + 1, 1)
            wait_chunk(c0, 0)
            compute(c0, 0)
            nxt = jnp.minimum(c0 + 2, n_chunks - 2)
            issue(nxt, 0)
            wait_chunk(c0 + 1, 1)
            compute(c0 + 1, 1)
            return carry

        lax.fori_loop(0, n_chunks // 2, pair, 0)
        wait_chunk(n_chunks - 2, 0)
        pltpu.sync_copy(sv, out_hbm.at[pl.ds(base, per_w)])

    return scorer


def kernel(pos_h, pos_r, pos_t, neg_h, neg_r, neg_t,
           entity_head, entity_tail, relation_head, relation_tail):
    b = pos_h.shape[0]
    h = jnp.concatenate([pos_h, neg_h])
    t = jnp.concatenate([pos_t, neg_t])
    r = jnp.concatenate([pos_r, neg_r])
    epack = _make_tc_pack(entity_head.shape[0])
    rpack = _make_tc_pack(relation_head.shape[0])
    ent = epack(entity_head.T, entity_tail.T)
    rel = rpack(relation_head.T, relation_tail.T)
    scorer = _make_sc_scorer(2 * b)
    out = scorer(h, t, r, ent, rel)
    return out[:b], out[b:]
